# Initial kernel scaffold; baseline (speedup 1.0000x reference)
import jax
import jax.numpy as jnp
from jax.experimental import pallas as pl


def kernel(x, edge_index, batch, pair_feature, y, W_gcn0, b_gcn0, W_gcn1, b_gcn1, W_enc, b_enc, W_lin1, b_lin1, W_lin2, b_lin2, W_log1, b_log1, W_log2, b_log2):
    raise NotImplementedError("write your pallas kernel here")


# speedup vs baseline: 1.0000x; 1.0000x over previous
import jax
import jax.numpy as jnp
from jax.experimental import pallas as pl
from jax.experimental.pallas import tpu as pltpu

N_NODES = 10000
B = 64
ALPHA = 0.5
BETA = 0.5


def _enc_body(pf_ref, w_ref, b_ref, out_ref):
    out_ref[...] = jnp.tanh(
        jnp.dot(pf_ref[...], w_ref[...], preferred_element_type=jnp.float32)
        + b_ref[...]
    )


def _gcn_conv(h, edge_index, W, b):
    N = h.shape[0]
    src = edge_index[0]
    dst = edge_index[1]
    loop = jnp.arange(N, dtype=src.dtype)
    s = jnp.concatenate([src, loop])
    d = jnp.concatenate([dst, loop])
    h = h @ W
    deg = jnp.zeros((N,), dtype=h.dtype).at[d].add(1.0)
    dis = jnp.where(deg > 0, deg ** -0.5, 0.0)
    coef = dis[s] * dis[d]
    out = jnp.zeros((N, h.shape[1]), dtype=h.dtype).at[d].add(coef[:, None] * h[s])
    return out + b


def _bce(p, t):
    p = jnp.clip(p, 1e-7, 1.0 - 1e-7)
    return -jnp.mean(t * jnp.log(p) + (1.0 - t) * jnp.log(1.0 - p))


def kernel(x, edge_index, batch, pair_feature, y, W_gcn0, b_gcn0, W_gcn1, b_gcn1,
           W_enc, b_enc, W_lin1, b_lin1, W_lin2, b_lin2, W_log1, b_log1,
           W_log2, b_log2):
    hidden1 = pl.pallas_call(
        _enc_body,
        out_shape=jax.ShapeDtypeStruct((B, W_enc.shape[1]), jnp.float32),
    )(pair_feature, W_enc, b_enc)

    l = x
    cats = []
    l = jnp.tanh(_gcn_conv(l, edge_index, W_gcn0, b_gcn0))
    cats.append(l)
    l = jnp.tanh(_gcn_conv(l, edge_index, W_gcn1, b_gcn1))
    cats.append(l)
    layer2 = jnp.concatenate(cats, axis=1)
    idx = jnp.searchsorted(batch, jnp.arange(B, dtype=batch.dtype))
    layer2 = layer2[idx, :]
    hidden2 = layer2 @ W_lin1 + b_lin1
    hidden = jnp.concatenate([hidden1, hidden2], axis=1)
    hidden = jax.nn.relu(hidden)
    h1 = jax.nn.relu(hidden1)
    logits1 = jax.nn.sigmoid(h1 @ W_log1 + b_log1).squeeze(1)
    h2 = jax.nn.relu(hidden2)
    logits2 = jax.nn.sigmoid(h2 @ W_log2 + b_log2).squeeze(1)
    logits = jax.nn.sigmoid(hidden @ W_lin2 + b_lin2).squeeze(1)
    loss_1 = _bce(logits1, y)
    loss_2 = _bce(logits2, y)
    loss_3 = _bce(logits, y)
    loss = ALPHA * loss_1 + BETA * loss_2 + loss_3
    pred = logits > 0.7
    acc = jnp.mean((pred.astype(jnp.float32) == y).astype(jnp.float32))
    return (logits, loss, acc, loss_1, loss_2, loss_3, hidden2)


# speedup vs baseline: 20.4971x; 20.4967x over previous
import functools

import jax
import jax.numpy as jnp
from jax import lax
from jax.experimental import pallas as pl
from jax.experimental.pallas import tpu as pltpu
from jax.experimental.pallas import tpu_sc as plsc

N_NODES = 10000
N_EDGES = 320000
D_FEAT = 128
D = 128
B = 64
ALPHA = 0.5
BETA = 0.5

NC = 2
NS = 16
NW = NC * NS
EDGES_PER_TILE = N_EDGES // NW
AGG_WIN = 200
AGG_NWIN = EDGES_PER_TILE // AGG_WIN
DEG_WIN = 2000
DEG_NWIN = EDGES_PER_TILE // DEG_WIN
N_PAD = 10240
ROWS_PER_TILE = N_NODES // NS
PAD_PER_TILE = N_PAD // NS


def _sc_mesh():
    return plsc.VectorSubcoreMesh(core_axis_name="c", subcore_axis_name="s",
                                  num_cores=NC, num_subcores=NS)


@functools.partial(
    pl.kernel,
    out_type=jax.ShapeDtypeStruct((NC, N_PAD), jnp.float32),
    mesh=_sc_mesh(),
    scratch_types=[
        pltpu.VMEM((DEG_WIN,), jnp.int32),
        pltpu.VMEM((DEG_WIN,), jnp.float32),
        pltpu.VMEM_SHARED((N_PAD,), jnp.float32),
        pltpu.SemaphoreType.DMA,
    ],
)
def _sc_deg(dst_hbm, ones_hbm, zeros_hbm, out_hbm, dst_v, ones_v, acc_sh, sem):
    cid = lax.axis_index("c")
    sid = lax.axis_index("s")
    wid = sid * NC + cid
    zbase = sid * PAD_PER_TILE
    pltpu.sync_copy(zeros_hbm.at[pl.ds(zbase, PAD_PER_TILE)],
                    acc_sh.at[pl.ds(zbase, PAD_PER_TILE)])
    pltpu.sync_copy(ones_hbm, ones_v)
    plsc.subcore_barrier()
    e_base = wid * EDGES_PER_TILE

    def body(w, _):
        off = e_base + w * DEG_WIN
        pltpu.sync_copy(dst_hbm.at[pl.ds(off, DEG_WIN)], dst_v)
        pltpu.sync_copy(ones_v, acc_sh.at[dst_v], add=True)
        return 0

    lax.fori_loop(0, DEG_NWIN, body, 0)
    plsc.subcore_barrier()
    pltpu.sync_copy(acc_sh.at[pl.ds(zbase, PAD_PER_TILE)],
                    out_hbm.at[cid, pl.ds(zbase, PAD_PER_TILE)])


@functools.partial(
    pl.kernel,
    out_type=jax.ShapeDtypeStruct((NC, N_PAD, D), jnp.float32),
    mesh=_sc_mesh(),
    scratch_types=[
        pltpu.VMEM((AGG_WIN,), jnp.int32),
        pltpu.VMEM((AGG_WIN,), jnp.int32),
        pltpu.VMEM((AGG_WIN, D), jnp.float32),
        pltpu.VMEM_SHARED((N_PAD, D), jnp.float32),
        pltpu.SemaphoreType.DMA,
    ],
)
def _sc_agg(g_hbm, src_hbm, dst_hbm, zrows_hbm, out_hbm,
            src_v, dst_v, rows_v, acc_sh, sem):
    cid = lax.axis_index("c")
    sid = lax.axis_index("s")
    wid = sid * NC + cid
    rbase = sid * PAD_PER_TILE
    pltpu.sync_copy(zrows_hbm.at[pl.ds(rbase, PAD_PER_TILE)],
                    acc_sh.at[pl.ds(rbase, PAD_PER_TILE)])
    plsc.subcore_barrier()
    e_base = wid * EDGES_PER_TILE

    def body(w, _):
        off = e_base + w * AGG_WIN
        pltpu.sync_copy(src_hbm.at[pl.ds(off, AGG_WIN)], src_v)
        pltpu.sync_copy(dst_hbm.at[pl.ds(off, AGG_WIN)], dst_v)
        pltpu.async_copy(g_hbm.at[src_v], rows_v, sem).wait()
        pltpu.sync_copy(rows_v, acc_sh.at[dst_v], add=True)
        return 0

    lax.fori_loop(0, AGG_NWIN, body, 0)
    plsc.subcore_barrier()
    pltpu.sync_copy(acc_sh.at[pl.ds(rbase, PAD_PER_TILE)],
                    out_hbm.at[cid, pl.ds(rbase, PAD_PER_TILE)])


def _tc_prep_body(x_ref, w_ref, dega_ref, degb_ref, g_ref, dis_ref):
    deg = dega_ref[0, :] + degb_ref[0, :] + 1.0
    dis = lax.rsqrt(deg)
    dis_ref[0, :] = dis
    h = jnp.dot(x_ref[...], w_ref[...], preferred_element_type=jnp.float32)
    g_ref[...] = h * dis[:, None]


def _tc_mid_body(acca_ref, accb_ref, g0_ref, dis_ref, b0_ref, w1_ref,
                 l1_ref, g1_ref):
    dis = dis_ref[0, :]
    agg = acca_ref[:N_NODES, :] + accb_ref[:N_NODES, :] + g0_ref[...]
    l1 = jnp.tanh(agg * dis[:, None] + b0_ref[...])
    l1_ref[...] = l1
    h1 = jnp.dot(l1, w1_ref[...], preferred_element_type=jnp.float32)
    g1_ref[...] = h1 * dis[:, None]


def _bce_terms(p, t):
    p = jnp.clip(p, 1e-7, 1.0 - 1e-7)
    return -jnp.mean(t * jnp.log(p) + (1.0 - t) * jnp.log(1.0 - p))


def _tc_head_body(acca_ref, accb_ref, g1_ref, dis_ref, b1_ref, l1_ref,
                  batch_ref, pf_ref, y_ref,
                  wenc_ref, benc_ref, wlin1_ref, blin1_ref, wlin2_ref,
                  blin2_ref, wlog1_ref, blog1_ref, wlog2_ref, blog2_ref,
                  logits_ref, loss_ref, acc_out_ref, l1o_ref, l2o_ref,
                  l3o_ref, hidden2_ref):
    dis = dis_ref[0, :]
    agg = acca_ref[:N_NODES, :] + accb_ref[:N_NODES, :] + g1_ref[...]
    l2 = jnp.tanh(agg * dis[:, None] + b1_ref[...])

    batch = batch_ref[0, :]
    brange = lax.broadcasted_iota(jnp.int32, (B, N_NODES), 0)
    idx = jnp.sum((batch[None, :] < brange).astype(jnp.int32), axis=1)
    node_iota = lax.broadcasted_iota(jnp.int32, (B, N_NODES), 1)
    onehot = (node_iota == idx[:, None]).astype(jnp.float32)
    sel1 = jnp.dot(onehot, l1_ref[...], preferred_element_type=jnp.float32)
    sel2 = jnp.dot(onehot, l2, preferred_element_type=jnp.float32)
    layer2 = jnp.concatenate([sel1, sel2], axis=1)

    hidden1 = jnp.tanh(
        jnp.dot(pf_ref[...], wenc_ref[...], preferred_element_type=jnp.float32)
        + benc_ref[...])
    hidden2 = jnp.dot(layer2, wlin1_ref[...],
                      preferred_element_type=jnp.float32) + blin1_ref[...]
    hidden2_ref[...] = hidden2
    hidden = jax.nn.relu(jnp.concatenate([hidden1, hidden2], axis=1))
    h1 = jax.nn.relu(hidden1)
    h2 = jax.nn.relu(hidden2)
    y = y_ref[0, :]
    logits1 = jax.nn.sigmoid(
        jnp.dot(h1, wlog1_ref[...], preferred_element_type=jnp.float32)
        + blog1_ref[...])[:, 0]
    logits2 = jax.nn.sigmoid(
        jnp.dot(h2, wlog2_ref[...], preferred_element_type=jnp.float32)
        + blog2_ref[...])[:, 0]
    logits = jax.nn.sigmoid(
        jnp.dot(hidden, wlin2_ref[...], preferred_element_type=jnp.float32)
        + blin2_ref[...])[:, 0]
    loss_1 = _bce_terms(logits1, y)
    loss_2 = _bce_terms(logits2, y)
    loss_3 = _bce_terms(logits, y)
    logits_ref[...] = logits.reshape(1, B)
    l1o_ref[...] = loss_1.reshape(1, 1)
    l2o_ref[...] = loss_2.reshape(1, 1)
    l3o_ref[...] = loss_3.reshape(1, 1)
    loss_ref[...] = (ALPHA * loss_1 + BETA * loss_2 + loss_3).reshape(1, 1)
    pred = (logits > 0.7).astype(jnp.float32)
    acc_out_ref[...] = jnp.mean((pred == y).astype(jnp.float32)).reshape(1, 1)


def kernel(x, edge_index, batch, pair_feature, y, W_gcn0, b_gcn0, W_gcn1,
           b_gcn1, W_enc, b_enc, W_lin1, b_lin1, W_lin2, b_lin2, W_log1,
           b_log1, W_log2, b_log2):
    src = edge_index[0].astype(jnp.int32)
    dst = edge_index[1].astype(jnp.int32)
    batch = batch.astype(jnp.int32)
    ones_win = jnp.ones((DEG_WIN,), jnp.float32)
    zeros_pad = jnp.zeros((N_PAD,), jnp.float32)
    zeros_rows = jnp.zeros((N_PAD, D), jnp.float32)

    degp = _sc_deg(dst, ones_win, zeros_pad)

    g0, dis = pl.pallas_call(
        _tc_prep_body,
        out_shape=[
            jax.ShapeDtypeStruct((N_NODES, D), jnp.float32),
            jax.ShapeDtypeStruct((1, N_NODES), jnp.float32),
        ],
    )(x, W_gcn0, degp[0:1, :N_NODES], degp[1:2, :N_NODES])

    acc0 = _sc_agg(g0, src, dst, zeros_rows)

    l1, g1 = pl.pallas_call(
        _tc_mid_body,
        out_shape=[
            jax.ShapeDtypeStruct((N_NODES, D), jnp.float32),
            jax.ShapeDtypeStruct((N_NODES, D), jnp.float32),
        ],
    )(acc0[0], acc0[1], g0, dis, b_gcn0.reshape(1, D), W_gcn1)

    acc1 = _sc_agg(g1, src, dst, zeros_rows)

    outs = pl.pallas_call(
        _tc_head_body,
        out_shape=[
            jax.ShapeDtypeStruct((1, B), jnp.float32),
            jax.ShapeDtypeStruct((1, 1), jnp.float32),
            jax.ShapeDtypeStruct((1, 1), jnp.float32),
            jax.ShapeDtypeStruct((1, 1), jnp.float32),
            jax.ShapeDtypeStruct((1, 1), jnp.float32),
            jax.ShapeDtypeStruct((1, 1), jnp.float32),
            jax.ShapeDtypeStruct((B, D), jnp.float32),
        ],
    )(acc1[0], acc1[1], g1, dis, b_gcn1.reshape(1, D), l1,
      batch.reshape(1, N_NODES), pair_feature, y.reshape(1, B),
      W_enc, b_enc.reshape(1, -1), W_lin1, b_lin1.reshape(1, -1),
      W_lin2, b_lin2.reshape(1, -1), W_log1, b_log1.reshape(1, -1),
      W_log2, b_log2.reshape(1, -1))

    logits, loss, acc, loss_1, loss_2, loss_3, hidden2 = outs
    return (logits.reshape(B), loss.reshape(()), acc.reshape(()),
            loss_1.reshape(()), loss_2.reshape(()), loss_3.reshape(()),
            hidden2)


# speedup vs baseline: 21.8976x; 1.0683x over previous
import functools

import jax
import jax.numpy as jnp
from jax import lax
from jax.experimental import pallas as pl
from jax.experimental.pallas import tpu as pltpu
from jax.experimental.pallas import tpu_sc as plsc

N_NODES = 10000
N_EDGES = 320000
D_FEAT = 128
D = 128
B = 64
ALPHA = 0.5
BETA = 0.5

NC = 2
NS = 16
NW = NC * NS
EDGES_PER_TILE = N_EDGES // NW
AGG_WIN = 80
AGG_NWIN = EDGES_PER_TILE // AGG_WIN
DEG_WIN = 2000
DEG_NWIN = EDGES_PER_TILE // DEG_WIN
N_PAD = 10240
ROWS_PER_TILE = N_NODES // NS
PAD_PER_TILE = N_PAD // NS


def _sc_mesh():
    return plsc.VectorSubcoreMesh(core_axis_name="c", subcore_axis_name="s",
                                  num_cores=NC, num_subcores=NS)


@functools.partial(
    pl.kernel,
    out_type=jax.ShapeDtypeStruct((NC, N_PAD), jnp.float32),
    mesh=_sc_mesh(),
    scratch_types=[
        pltpu.VMEM((DEG_WIN,), jnp.int32),
        pltpu.VMEM((DEG_WIN,), jnp.float32),
        pltpu.VMEM_SHARED((N_PAD,), jnp.float32),
        pltpu.SemaphoreType.DMA,
    ],
)
def _sc_deg(dst_hbm, ones_hbm, zeros_hbm, out_hbm, dst_v, ones_v, acc_sh, sem):
    cid = lax.axis_index("c")
    sid = lax.axis_index("s")
    wid = sid * NC + cid
    zbase = sid * PAD_PER_TILE
    pltpu.sync_copy(zeros_hbm.at[pl.ds(zbase, PAD_PER_TILE)],
                    acc_sh.at[pl.ds(zbase, PAD_PER_TILE)])
    pltpu.sync_copy(ones_hbm, ones_v)
    plsc.subcore_barrier()
    e_base = wid * EDGES_PER_TILE

    def body(w, _):
        off = e_base + w * DEG_WIN
        pltpu.sync_copy(dst_hbm.at[pl.ds(off, DEG_WIN)], dst_v)
        pltpu.sync_copy(ones_v, acc_sh.at[dst_v], add=True)
        return 0

    lax.fori_loop(0, DEG_NWIN, body, 0)
    plsc.subcore_barrier()
    pltpu.sync_copy(acc_sh.at[pl.ds(zbase, PAD_PER_TILE)],
                    out_hbm.at[cid, pl.ds(zbase, PAD_PER_TILE)])


@functools.partial(
    pl.kernel,
    out_type=jax.ShapeDtypeStruct((NC, N_PAD, D), jnp.float32),
    mesh=_sc_mesh(),
    scratch_types=[
        pltpu.VMEM((AGG_WIN,), jnp.int32),
        pltpu.VMEM((AGG_WIN,), jnp.int32),
        pltpu.VMEM((AGG_WIN,), jnp.int32),
        pltpu.VMEM((AGG_WIN,), jnp.int32),
        pltpu.VMEM((AGG_WIN, D), jnp.float32),
        pltpu.VMEM((AGG_WIN, D), jnp.float32),
        pltpu.VMEM_SHARED((N_PAD, D), jnp.float32),
        pltpu.SemaphoreType.DMA,
        pltpu.SemaphoreType.DMA,
    ],
)
def _sc_agg(g_hbm, src_hbm, dst_hbm, zrows_hbm, out_hbm,
            src0_v, dst0_v, src1_v, dst1_v, rows0_v, rows1_v, acc_sh,
            sem0, sem1):
    cid = lax.axis_index("c")
    sid = lax.axis_index("s")
    wid = sid * NC + cid
    rbase = sid * PAD_PER_TILE
    pltpu.sync_copy(zrows_hbm.at[pl.ds(rbase, PAD_PER_TILE)],
                    acc_sh.at[pl.ds(rbase, PAD_PER_TILE)])
    plsc.subcore_barrier()
    e_base = wid * EDGES_PER_TILE

    def fetch(w, src_v, dst_v, rows_v, sem):
        off = e_base + w * AGG_WIN
        pltpu.sync_copy(src_hbm.at[pl.ds(off, AGG_WIN)], src_v)
        pltpu.sync_copy(dst_hbm.at[pl.ds(off, AGG_WIN)], dst_v)
        return pltpu.async_copy(g_hbm.at[src_v], rows_v, sem)

    fetch(0, src0_v, dst0_v, rows0_v, sem0)

    def body(p, _):
        wa = 2 * p + 1
        fetch(wa, src1_v, dst1_v, rows1_v, sem1)
        pltpu.make_async_copy(g_hbm.at[src0_v], rows0_v, sem0).wait()
        pltpu.sync_copy(rows0_v, acc_sh.at[dst0_v], add=True)
        fetch(wa + 1, src0_v, dst0_v, rows0_v, sem0)
        pltpu.make_async_copy(g_hbm.at[src1_v], rows1_v, sem1).wait()
        pltpu.sync_copy(rows1_v, acc_sh.at[dst1_v], add=True)
        return 0

    lax.fori_loop(0, (AGG_NWIN - 1) // 2, body, 0)
    pltpu.make_async_copy(g_hbm.at[src0_v], rows0_v, sem0).wait()
    pltpu.sync_copy(rows0_v, acc_sh.at[dst0_v], add=True)
    plsc.subcore_barrier()
    pltpu.sync_copy(acc_sh.at[pl.ds(rbase, PAD_PER_TILE)],
                    out_hbm.at[cid, pl.ds(rbase, PAD_PER_TILE)])


def _tc_prep_body(x_ref, w_ref, dega_ref, degb_ref, g_ref, dis_ref):
    deg = dega_ref[0, :] + degb_ref[0, :] + 1.0
    dis = lax.rsqrt(deg)
    dis_ref[0, :] = dis
    h = jnp.dot(x_ref[...], w_ref[...], preferred_element_type=jnp.float32)
    g_ref[...] = h * dis[:, None]


def _tc_mid_body(acca_ref, accb_ref, g0_ref, dis_ref, b0_ref, w1_ref,
                 l1_ref, g1_ref):
    dis = dis_ref[0, :]
    agg = acca_ref[:N_NODES, :] + accb_ref[:N_NODES, :] + g0_ref[...]
    l1 = jnp.tanh(agg * dis[:, None] + b0_ref[...])
    l1_ref[...] = l1
    h1 = jnp.dot(l1, w1_ref[...], preferred_element_type=jnp.float32)
    g1_ref[...] = h1 * dis[:, None]


def _bce_terms(p, t):
    p = jnp.clip(p, 1e-7, 1.0 - 1e-7)
    return -jnp.mean(t * jnp.log(p) + (1.0 - t) * jnp.log(1.0 - p))


def _tc_head_body(acca_ref, accb_ref, g1_ref, dis_ref, b1_ref, l1_ref,
                  batch_ref, pf_ref, y_ref,
                  wenc_ref, benc_ref, wlin1_ref, blin1_ref, wlin2_ref,
                  blin2_ref, wlog1_ref, blog1_ref, wlog2_ref, blog2_ref,
                  logits_ref, loss_ref, acc_out_ref, l1o_ref, l2o_ref,
                  l3o_ref, hidden2_ref):
    dis = dis_ref[0, :]
    agg = acca_ref[:N_NODES, :] + accb_ref[:N_NODES, :] + g1_ref[...]
    l2 = jnp.tanh(agg * dis[:, None] + b1_ref[...])

    batch = batch_ref[0, :]
    brange = lax.broadcasted_iota(jnp.int32, (B, N_NODES), 0)
    idx = jnp.sum((batch[None, :] < brange).astype(jnp.int32), axis=1)
    node_iota = lax.broadcasted_iota(jnp.int32, (B, N_NODES), 1)
    onehot = (node_iota == idx[:, None]).astype(jnp.float32)
    sel1 = jnp.dot(onehot, l1_ref[...], preferred_element_type=jnp.float32)
    sel2 = jnp.dot(onehot, l2, preferred_element_type=jnp.float32)
    layer2 = jnp.concatenate([sel1, sel2], axis=1)

    hidden1 = jnp.tanh(
        jnp.dot(pf_ref[...], wenc_ref[...], preferred_element_type=jnp.float32)
        + benc_ref[...])
    hidden2 = jnp.dot(layer2, wlin1_ref[...],
                      preferred_element_type=jnp.float32) + blin1_ref[...]
    hidden2_ref[...] = hidden2
    hidden = jax.nn.relu(jnp.concatenate([hidden1, hidden2], axis=1))
    h1 = jax.nn.relu(hidden1)
    h2 = jax.nn.relu(hidden2)
    y = y_ref[0, :]
    logits1 = jax.nn.sigmoid(
        jnp.dot(h1, wlog1_ref[...], preferred_element_type=jnp.float32)
        + blog1_ref[...])[:, 0]
    logits2 = jax.nn.sigmoid(
        jnp.dot(h2, wlog2_ref[...], preferred_element_type=jnp.float32)
        + blog2_ref[...])[:, 0]
    logits = jax.nn.sigmoid(
        jnp.dot(hidden, wlin2_ref[...], preferred_element_type=jnp.float32)
        + blin2_ref[...])[:, 0]
    loss_1 = _bce_terms(logits1, y)
    loss_2 = _bce_terms(logits2, y)
    loss_3 = _bce_terms(logits, y)
    logits_ref[...] = logits.reshape(1, B)
    l1o_ref[...] = loss_1.reshape(1, 1)
    l2o_ref[...] = loss_2.reshape(1, 1)
    l3o_ref[...] = loss_3.reshape(1, 1)
    loss_ref[...] = (ALPHA * loss_1 + BETA * loss_2 + loss_3).reshape(1, 1)
    pred = (logits > 0.7).astype(jnp.float32)
    acc_out_ref[...] = jnp.mean((pred == y).astype(jnp.float32)).reshape(1, 1)


def kernel(x, edge_index, batch, pair_feature, y, W_gcn0, b_gcn0, W_gcn1,
           b_gcn1, W_enc, b_enc, W_lin1, b_lin1, W_lin2, b_lin2, W_log1,
           b_log1, W_log2, b_log2):
    src = edge_index[0].astype(jnp.int32)
    dst = edge_index[1].astype(jnp.int32)
    batch = batch.astype(jnp.int32)
    ones_win = jnp.ones((DEG_WIN,), jnp.float32)
    zeros_pad = jnp.zeros((N_PAD,), jnp.float32)
    zeros_rows = jnp.zeros((N_PAD, D), jnp.float32)

    degp = _sc_deg(dst, ones_win, zeros_pad)

    g0, dis = pl.pallas_call(
        _tc_prep_body,
        out_shape=[
            jax.ShapeDtypeStruct((N_NODES, D), jnp.float32),
            jax.ShapeDtypeStruct((1, N_NODES), jnp.float32),
        ],
    )(x, W_gcn0, degp[0:1, :N_NODES], degp[1:2, :N_NODES])

    acc0 = _sc_agg(g0, src, dst, zeros_rows)

    l1, g1 = pl.pallas_call(
        _tc_mid_body,
        out_shape=[
            jax.ShapeDtypeStruct((N_NODES, D), jnp.float32),
            jax.ShapeDtypeStruct((N_NODES, D), jnp.float32),
        ],
    )(acc0[0], acc0[1], g0, dis, b_gcn0.reshape(1, D), W_gcn1)

    acc1 = _sc_agg(g1, src, dst, zeros_rows)

    outs = pl.pallas_call(
        _tc_head_body,
        out_shape=[
            jax.ShapeDtypeStruct((1, B), jnp.float32),
            jax.ShapeDtypeStruct((1, 1), jnp.float32),
            jax.ShapeDtypeStruct((1, 1), jnp.float32),
            jax.ShapeDtypeStruct((1, 1), jnp.float32),
            jax.ShapeDtypeStruct((1, 1), jnp.float32),
            jax.ShapeDtypeStruct((1, 1), jnp.float32),
            jax.ShapeDtypeStruct((B, D), jnp.float32),
        ],
    )(acc1[0], acc1[1], g1, dis, b_gcn1.reshape(1, D), l1,
      batch.reshape(1, N_NODES), pair_feature, y.reshape(1, B),
      W_enc, b_enc.reshape(1, -1), W_lin1, b_lin1.reshape(1, -1),
      W_lin2, b_lin2.reshape(1, -1), W_log1, b_log1.reshape(1, -1),
      W_log2, b_log2.reshape(1, -1))

    logits, loss, acc, loss_1, loss_2, loss_3, hidden2 = outs
    return (logits.reshape(B), loss.reshape(()), acc.reshape(()),
            loss_1.reshape(()), loss_2.reshape(()), loss_3.reshape(()),
            hidden2)


# speedup vs baseline: 27.5579x; 1.2585x over previous
import functools

import jax
import jax.numpy as jnp
from jax import lax
from jax.experimental import pallas as pl
from jax.experimental.pallas import tpu as pltpu
from jax.experimental.pallas import tpu_sc as plsc

N_NODES = 10000
N_EDGES = 320000
D_FEAT = 128
D = 128
B = 64
ALPHA = 0.5
BETA = 0.5

NC = 2
NS = 16
NW = NC * NS
EDGES_PER_TILE = N_EDGES // NW
PAD_E = 80
EPT_P = EDGES_PER_TILE + PAD_E
AGG_WIN = 160
AGG_NWIN = EPT_P // AGG_WIN
DEG_WIN = 2016
DEG_NWIN = EPT_P // DEG_WIN
N_PAD = 10240
ROWS_PER_TILE = N_NODES // NS
PAD_PER_TILE = N_PAD // NS


def _sc_mesh():
    return plsc.VectorSubcoreMesh(core_axis_name="c", subcore_axis_name="s",
                                  num_cores=NC, num_subcores=NS)


@functools.partial(
    pl.kernel,
    out_type=jax.ShapeDtypeStruct((NC, N_PAD), jnp.float32),
    mesh=_sc_mesh(),
    scratch_types=[
        pltpu.VMEM((DEG_WIN,), jnp.int32),
        pltpu.VMEM((DEG_WIN,), jnp.float32),
        pltpu.VMEM_SHARED((N_PAD,), jnp.float32),
        pltpu.SemaphoreType.DMA,
    ],
)
def _sc_deg(dst_hbm, ones_hbm, zeros_hbm, out_hbm, dst_v, ones_v, acc_sh, sem):
    cid = lax.axis_index("c")
    sid = lax.axis_index("s")
    wid = sid * NC + cid
    zbase = sid * PAD_PER_TILE
    pltpu.sync_copy(zeros_hbm.at[pl.ds(zbase, PAD_PER_TILE)],
                    acc_sh.at[pl.ds(zbase, PAD_PER_TILE)])
    pltpu.sync_copy(ones_hbm, ones_v)
    plsc.subcore_barrier()
    e_base = wid * EPT_P

    def body(w, _):
        off = e_base + w * DEG_WIN
        pltpu.sync_copy(dst_hbm.at[pl.ds(off, DEG_WIN)], dst_v)
        pltpu.sync_copy(ones_v, acc_sh.at[dst_v], add=True)
        return 0

    lax.fori_loop(0, DEG_NWIN, body, 0)
    plsc.subcore_barrier()
    pltpu.sync_copy(acc_sh.at[pl.ds(zbase, PAD_PER_TILE)],
                    out_hbm.at[cid, pl.ds(zbase, PAD_PER_TILE)])


@functools.partial(
    pl.kernel,
    out_type=jax.ShapeDtypeStruct((NC, N_PAD, D), jnp.float32),
    mesh=_sc_mesh(),
    scratch_types=[
        pltpu.VMEM((AGG_WIN,), jnp.int32),
        pltpu.VMEM((AGG_WIN,), jnp.int32),
        pltpu.VMEM((AGG_WIN,), jnp.int32),
        pltpu.VMEM((AGG_WIN,), jnp.int32),
        pltpu.VMEM((AGG_WIN, D), jnp.float32),
        pltpu.VMEM((AGG_WIN, D), jnp.float32),
        pltpu.VMEM_SHARED((N_PAD, D), jnp.float32),
        pltpu.SemaphoreType.DMA,
        pltpu.SemaphoreType.DMA,
    ],
)
def _sc_agg(g_hbm, src_hbm, dst_hbm, zrows_hbm, out_hbm,
            src0_v, dst0_v, src1_v, dst1_v, rows0_v, rows1_v, acc_sh,
            sem0, sem1):
    cid = lax.axis_index("c")
    sid = lax.axis_index("s")
    wid = sid * NC + cid
    rbase = sid * PAD_PER_TILE
    pltpu.sync_copy(zrows_hbm.at[pl.ds(rbase, PAD_PER_TILE)],
                    acc_sh.at[pl.ds(rbase, PAD_PER_TILE)])
    plsc.subcore_barrier()
    e_base = wid * EPT_P

    def fetch(w, src_v, dst_v, rows_v, sem):
        off = e_base + w * AGG_WIN
        pltpu.sync_copy(src_hbm.at[pl.ds(off, AGG_WIN)], src_v)
        pltpu.sync_copy(dst_hbm.at[pl.ds(off, AGG_WIN)], dst_v)
        return pltpu.async_copy(g_hbm.at[src_v], rows_v, sem)

    fetch(0, src0_v, dst0_v, rows0_v, sem0)

    def body(p, _):
        wa = 2 * p + 1
        fetch(wa, src1_v, dst1_v, rows1_v, sem1)
        pltpu.make_async_copy(g_hbm.at[src0_v], rows0_v, sem0).wait()
        pltpu.sync_copy(rows0_v, acc_sh.at[dst0_v], add=True)
        fetch(wa + 1, src0_v, dst0_v, rows0_v, sem0)
        pltpu.make_async_copy(g_hbm.at[src1_v], rows1_v, sem1).wait()
        pltpu.sync_copy(rows1_v, acc_sh.at[dst1_v], add=True)
        return 0

    lax.fori_loop(0, (AGG_NWIN - 1) // 2, body, 0)
    pltpu.make_async_copy(g_hbm.at[src0_v], rows0_v, sem0).wait()
    pltpu.sync_copy(rows0_v, acc_sh.at[dst0_v], add=True)
    plsc.subcore_barrier()
    pltpu.sync_copy(acc_sh.at[pl.ds(rbase, PAD_PER_TILE)],
                    out_hbm.at[cid, pl.ds(rbase, PAD_PER_TILE)])


def _tc_prep_body(x_ref, w_ref, dega_ref, degb_ref, g_ref, dis_ref):
    deg = dega_ref[0, :] + degb_ref[0, :] + 1.0
    dis = lax.rsqrt(deg)
    dis_ref[0, :] = dis
    h = jnp.dot(x_ref[...], w_ref[...], preferred_element_type=jnp.float32)
    g_ref[0:N_NODES, :] = h * dis[:, None]


def _tc_mid_body(acca_ref, accb_ref, g0_ref, dis_ref, b0_ref, w1_ref,
                 l1_ref, g1_ref):
    dis = dis_ref[0, :]
    agg = acca_ref[:N_NODES, :] + accb_ref[:N_NODES, :] + g0_ref[:N_NODES, :]
    l1 = jnp.tanh(agg * dis[:, None] + b0_ref[...])
    l1_ref[...] = l1
    h1 = jnp.dot(l1, w1_ref[...], preferred_element_type=jnp.float32)
    g1_ref[0:N_NODES, :] = h1 * dis[:, None]


def _bce_terms(p, t):
    p = jnp.clip(p, 1e-7, 1.0 - 1e-7)
    return -jnp.mean(t * jnp.log(p) + (1.0 - t) * jnp.log(1.0 - p))


def _tc_head_body(acca_ref, accb_ref, g1_ref, dis_ref, b1_ref, l1_ref,
                  batch_ref, pf_ref, y_ref,
                  wenc_ref, benc_ref, wlin1_ref, blin1_ref, wlin2_ref,
                  blin2_ref, wlog1_ref, blog1_ref, wlog2_ref, blog2_ref,
                  logits_ref, loss_ref, acc_out_ref, l1o_ref, l2o_ref,
                  l3o_ref, hidden2_ref):
    dis = dis_ref[0, :]
    agg = acca_ref[:N_NODES, :] + accb_ref[:N_NODES, :] + g1_ref[:N_NODES, :]
    l2 = jnp.tanh(agg * dis[:, None] + b1_ref[...])

    batch = batch_ref[0, :]
    brange = lax.broadcasted_iota(jnp.int32, (B, N_NODES), 0)
    idx = jnp.sum((batch[None, :] < brange).astype(jnp.int32), axis=1)
    node_iota = lax.broadcasted_iota(jnp.int32, (B, N_NODES), 1)
    onehot = (node_iota == idx[:, None]).astype(jnp.float32)
    sel1 = jnp.dot(onehot, l1_ref[...], preferred_element_type=jnp.float32)
    sel2 = jnp.dot(onehot, l2, preferred_element_type=jnp.float32)
    layer2 = jnp.concatenate([sel1, sel2], axis=1)

    hidden1 = jnp.tanh(
        jnp.dot(pf_ref[...], wenc_ref[...], preferred_element_type=jnp.float32)
        + benc_ref[...])
    hidden2 = jnp.dot(layer2, wlin1_ref[...],
                      preferred_element_type=jnp.float32) + blin1_ref[...]
    hidden2_ref[...] = hidden2
    hidden = jax.nn.relu(jnp.concatenate([hidden1, hidden2], axis=1))
    h1 = jax.nn.relu(hidden1)
    h2 = jax.nn.relu(hidden2)
    y = y_ref[0, :]
    logits1 = jax.nn.sigmoid(
        jnp.dot(h1, wlog1_ref[...], preferred_element_type=jnp.float32)
        + blog1_ref[...])[:, 0]
    logits2 = jax.nn.sigmoid(
        jnp.dot(h2, wlog2_ref[...], preferred_element_type=jnp.float32)
        + blog2_ref[...])[:, 0]
    logits = jax.nn.sigmoid(
        jnp.dot(hidden, wlin2_ref[...], preferred_element_type=jnp.float32)
        + blin2_ref[...])[:, 0]
    loss_1 = _bce_terms(logits1, y)
    loss_2 = _bce_terms(logits2, y)
    loss_3 = _bce_terms(logits, y)
    logits_ref[...] = logits.reshape(1, B)
    l1o_ref[...] = loss_1.reshape(1, 1)
    l2o_ref[...] = loss_2.reshape(1, 1)
    l3o_ref[...] = loss_3.reshape(1, 1)
    loss_ref[...] = (ALPHA * loss_1 + BETA * loss_2 + loss_3).reshape(1, 1)
    pred = (logits > 0.7).astype(jnp.float32)
    acc_out_ref[...] = jnp.mean((pred == y).astype(jnp.float32)).reshape(1, 1)


def kernel(x, edge_index, batch, pair_feature, y, W_gcn0, b_gcn0, W_gcn1,
           b_gcn1, W_enc, b_enc, W_lin1, b_lin1, W_lin2, b_lin2, W_log1,
           b_log1, W_log2, b_log2):
    src = edge_index[0].astype(jnp.int32)
    dst = edge_index[1].astype(jnp.int32)
    batch = batch.astype(jnp.int32)
    ones_win = jnp.ones((DEG_WIN,), jnp.float32)
    zeros_pad = jnp.zeros((N_PAD,), jnp.float32)
    zeros_rows = jnp.zeros((N_PAD, D), jnp.float32)

    pad_idx = (N_NODES + (jnp.arange(NW * PAD_E, dtype=jnp.int32)
                          % (N_PAD - N_NODES))).reshape(NW, PAD_E)
    src = jnp.concatenate([src.reshape(NW, EDGES_PER_TILE), pad_idx],
                          axis=1).reshape(-1)
    dst = jnp.concatenate([dst.reshape(NW, EDGES_PER_TILE), pad_idx],
                          axis=1).reshape(-1)

    degp = _sc_deg(dst, ones_win, zeros_pad)

    g0, dis = pl.pallas_call(
        _tc_prep_body,
        out_shape=[
            jax.ShapeDtypeStruct((N_PAD, D), jnp.float32),
            jax.ShapeDtypeStruct((1, N_NODES), jnp.float32),
        ],
    )(x, W_gcn0, degp[0:1, :N_NODES], degp[1:2, :N_NODES])

    acc0 = _sc_agg(g0, src, dst, zeros_rows)

    l1, g1 = pl.pallas_call(
        _tc_mid_body,
        out_shape=[
            jax.ShapeDtypeStruct((N_NODES, D), jnp.float32),
            jax.ShapeDtypeStruct((N_PAD, D), jnp.float32),
        ],
    )(acc0[0], acc0[1], g0, dis, b_gcn0.reshape(1, D), W_gcn1)

    acc1 = _sc_agg(g1, src, dst, zeros_rows)

    outs = pl.pallas_call(
        _tc_head_body,
        out_shape=[
            jax.ShapeDtypeStruct((1, B), jnp.float32),
            jax.ShapeDtypeStruct((1, 1), jnp.float32),
            jax.ShapeDtypeStruct((1, 1), jnp.float32),
            jax.ShapeDtypeStruct((1, 1), jnp.float32),
            jax.ShapeDtypeStruct((1, 1), jnp.float32),
            jax.ShapeDtypeStruct((1, 1), jnp.float32),
            jax.ShapeDtypeStruct((B, D), jnp.float32),
        ],
    )(acc1[0], acc1[1], g1, dis, b_gcn1.reshape(1, D), l1,
      batch.reshape(1, N_NODES), pair_feature, y.reshape(1, B),
      W_enc, b_enc.reshape(1, -1), W_lin1, b_lin1.reshape(1, -1),
      W_lin2, b_lin2.reshape(1, -1), W_log1, b_log1.reshape(1, -1),
      W_log2, b_log2.reshape(1, -1))

    logits, loss, acc, loss_1, loss_2, loss_3, hidden2 = outs
    return (logits.reshape(B), loss.reshape(()), acc.reshape(()),
            loss_1.reshape(()), loss_2.reshape(()), loss_3.reshape(()),
            hidden2)


# speedup vs baseline: 31.6937x; 1.1501x over previous
import functools

import jax
import jax.numpy as jnp
from jax import lax
from jax.experimental import pallas as pl
from jax.experimental.pallas import tpu as pltpu
from jax.experimental.pallas import tpu_sc as plsc

N_NODES = 10000
N_EDGES = 320000
D_FEAT = 128
D = 128
B = 64
ALPHA = 0.5
BETA = 0.5

NC = 2
NS = 16
NW = NC * NS
EDGES_PER_TILE = N_EDGES // NW
PAD_E = 224
EPT_P = EDGES_PER_TILE + PAD_E
AGG_WIN = 144
AGG_NWIN = EPT_P // AGG_WIN
DEG_WIN = 2044 // 4 * 4
DEG_NWIN = 0
DEG_WIN = 1704
DEG_NWIN = EPT_P // DEG_WIN
N_PAD = 10240
ROWS_PER_TILE = N_NODES // NS
PAD_PER_TILE = N_PAD // NS


def _sc_mesh():
    return plsc.VectorSubcoreMesh(core_axis_name="c", subcore_axis_name="s",
                                  num_cores=NC, num_subcores=NS)


@functools.partial(
    pl.kernel,
    out_type=jax.ShapeDtypeStruct((NC, N_PAD), jnp.float32),
    mesh=_sc_mesh(),
    scratch_types=[
        pltpu.VMEM((DEG_WIN,), jnp.int32),
        pltpu.VMEM((DEG_WIN,), jnp.float32),
        pltpu.VMEM_SHARED((N_PAD,), jnp.float32),
        pltpu.SemaphoreType.DMA,
    ],
)
def _sc_deg(dst_hbm, ones_hbm, zeros_hbm, out_hbm, dst_v, ones_v, acc_sh, sem):
    cid = lax.axis_index("c")
    sid = lax.axis_index("s")
    wid = sid * NC + cid
    zbase = sid * PAD_PER_TILE
    pltpu.sync_copy(zeros_hbm.at[pl.ds(zbase, PAD_PER_TILE)],
                    acc_sh.at[pl.ds(zbase, PAD_PER_TILE)])
    pltpu.sync_copy(ones_hbm, ones_v)
    plsc.subcore_barrier()
    e_base = wid * EPT_P

    def body(w, _):
        off = e_base + w * DEG_WIN
        pltpu.sync_copy(dst_hbm.at[pl.ds(off, DEG_WIN)], dst_v)
        pltpu.sync_copy(ones_v, acc_sh.at[dst_v], add=True)
        return 0

    lax.fori_loop(0, DEG_NWIN, body, 0)
    plsc.subcore_barrier()
    pltpu.sync_copy(acc_sh.at[pl.ds(zbase, PAD_PER_TILE)],
                    out_hbm.at[cid, pl.ds(zbase, PAD_PER_TILE)])


@functools.partial(
    pl.kernel,
    out_type=jax.ShapeDtypeStruct((NC, N_PAD, D), jnp.float32),
    mesh=_sc_mesh(),
    scratch_types=[
        pltpu.VMEM((EPT_P,), jnp.int32),
        pltpu.VMEM((AGG_WIN,), jnp.int32),
        pltpu.VMEM((AGG_WIN,), jnp.int32),
        pltpu.VMEM((AGG_WIN, D), jnp.float32),
        pltpu.VMEM((AGG_WIN, D), jnp.float32),
        pltpu.VMEM_SHARED((N_PAD, D), jnp.float32),
        pltpu.SemaphoreType.DMA,
        pltpu.SemaphoreType.DMA,
    ],
)
def _sc_agg(g_hbm, src_hbm, dst_hbm, zrows_hbm, out_hbm,
            srcall_v, dst0_v, dst1_v, rows0_v, rows1_v, acc_sh,
            sem0, sem1):
    cid = lax.axis_index("c")
    sid = lax.axis_index("s")
    wid = sid * NC + cid
    rbase = sid * PAD_PER_TILE
    pltpu.sync_copy(zrows_hbm.at[pl.ds(rbase, PAD_PER_TILE)],
                    acc_sh.at[pl.ds(rbase, PAD_PER_TILE)])
    e_base = wid * EPT_P
    pltpu.sync_copy(src_hbm.at[pl.ds(e_base, EPT_P)], srcall_v)
    plsc.subcore_barrier()

    def fetch(w, dst_v, rows_v, sem):
        off = e_base + w * AGG_WIN
        pltpu.sync_copy(dst_hbm.at[pl.ds(off, AGG_WIN)], dst_v)
        return pltpu.async_copy(
            g_hbm.at[srcall_v.at[pl.ds(w * AGG_WIN, AGG_WIN)]], rows_v, sem)

    fetch(0, dst0_v, rows0_v, sem0)

    def body(p, _):
        wa = 2 * p + 1
        fetch(wa, dst1_v, rows1_v, sem1)
        pltpu.make_async_copy(g_hbm.at[dst0_v], rows0_v, sem0).wait()
        pltpu.sync_copy(rows0_v, acc_sh.at[dst0_v], add=True)
        fetch(wa + 1, dst0_v, rows0_v, sem0)
        pltpu.make_async_copy(g_hbm.at[dst1_v], rows1_v, sem1).wait()
        pltpu.sync_copy(rows1_v, acc_sh.at[dst1_v], add=True)
        return 0

    lax.fori_loop(0, (AGG_NWIN - 1) // 2, body, 0)
    pltpu.make_async_copy(g_hbm.at[dst0_v], rows0_v, sem0).wait()
    pltpu.sync_copy(rows0_v, acc_sh.at[dst0_v], add=True)
    plsc.subcore_barrier()
    pltpu.sync_copy(acc_sh.at[pl.ds(rbase, PAD_PER_TILE)],
                    out_hbm.at[cid, pl.ds(rbase, PAD_PER_TILE)])


def _tc_prep_body(x_ref, w_ref, deg_ref, g_ref, dis_ref):
    deg = deg_ref[0, :N_NODES] + deg_ref[1, :N_NODES] + 1.0
    dis = lax.rsqrt(deg)
    dis_ref[0, :] = dis
    h = jnp.dot(x_ref[...], w_ref[...], preferred_element_type=jnp.float32)
    g_ref[0:N_NODES, :] = h * dis[:, None]


def _tc_mid_body(acc_ref, g0_ref, dis_ref, b0_ref, w1_ref,
                 l1_ref, g1_ref):
    dis = dis_ref[0, :]
    agg = (acc_ref[0, :N_NODES, :] + acc_ref[1, :N_NODES, :]
           + g0_ref[:N_NODES, :])
    l1 = jnp.tanh(agg * dis[:, None] + b0_ref[...])
    l1_ref[...] = l1
    h1 = jnp.dot(l1, w1_ref[...], preferred_element_type=jnp.float32)
    g1_ref[0:N_NODES, :] = h1 * dis[:, None]


def _bce_terms(p, t):
    p = jnp.clip(p, 1e-7, 1.0 - 1e-7)
    return -jnp.mean(t * jnp.log(p) + (1.0 - t) * jnp.log(1.0 - p))


def _tc_head_body(acc_ref, g1_ref, dis_ref, b1_ref, l1_ref,
                  batch_ref, pf_ref, y_ref,
                  wenc_ref, benc_ref, wlin1_ref, blin1_ref, wlin2_ref,
                  blin2_ref, wlog1_ref, blog1_ref, wlog2_ref, blog2_ref,
                  logits_ref, loss_ref, acc_out_ref, l1o_ref, l2o_ref,
                  l3o_ref, hidden2_ref):
    dis = dis_ref[0, :]
    agg = (acc_ref[0, :N_NODES, :] + acc_ref[1, :N_NODES, :]
           + g1_ref[:N_NODES, :])
    l2 = jnp.tanh(agg * dis[:, None] + b1_ref[...])

    batch = batch_ref[0, :]
    brange = lax.broadcasted_iota(jnp.int32, (B, N_NODES), 0)
    idx = jnp.sum((batch[None, :] < brange).astype(jnp.int32), axis=1)
    node_iota = lax.broadcasted_iota(jnp.int32, (B, N_NODES), 1)
    onehot = (node_iota == idx[:, None]).astype(jnp.float32)
    sel1 = jnp.dot(onehot, l1_ref[...], preferred_element_type=jnp.float32)
    sel2 = jnp.dot(onehot, l2, preferred_element_type=jnp.float32)
    layer2 = jnp.concatenate([sel1, sel2], axis=1)

    hidden1 = jnp.tanh(
        jnp.dot(pf_ref[...], wenc_ref[...], preferred_element_type=jnp.float32)
        + benc_ref[...])
    hidden2 = jnp.dot(layer2, wlin1_ref[...],
                      preferred_element_type=jnp.float32) + blin1_ref[...]
    hidden2_ref[...] = hidden2
    hidden = jax.nn.relu(jnp.concatenate([hidden1, hidden2], axis=1))
    h1 = jax.nn.relu(hidden1)
    h2 = jax.nn.relu(hidden2)
    y = y_ref[0, :]
    logits1 = jax.nn.sigmoid(
        jnp.dot(h1, wlog1_ref[...], preferred_element_type=jnp.float32)
        + blog1_ref[...])[:, 0]
    logits2 = jax.nn.sigmoid(
        jnp.dot(h2, wlog2_ref[...], preferred_element_type=jnp.float32)
        + blog2_ref[...])[:, 0]
    logits = jax.nn.sigmoid(
        jnp.dot(hidden, wlin2_ref[...], preferred_element_type=jnp.float32)
        + blin2_ref[...])[:, 0]
    loss_1 = _bce_terms(logits1, y)
    loss_2 = _bce_terms(logits2, y)
    loss_3 = _bce_terms(logits, y)
    logits_ref[...] = logits.reshape(1, B)
    l1o_ref[...] = loss_1.reshape(1, 1)
    l2o_ref[...] = loss_2.reshape(1, 1)
    l3o_ref[...] = loss_3.reshape(1, 1)
    loss_ref[...] = (ALPHA * loss_1 + BETA * loss_2 + loss_3).reshape(1, 1)
    pred = (logits > 0.7).astype(jnp.float32)
    acc_out_ref[...] = jnp.mean((pred == y).astype(jnp.float32)).reshape(1, 1)


def kernel(x, edge_index, batch, pair_feature, y, W_gcn0, b_gcn0, W_gcn1,
           b_gcn1, W_enc, b_enc, W_lin1, b_lin1, W_lin2, b_lin2, W_log1,
           b_log1, W_log2, b_log2):
    src = edge_index[0].astype(jnp.int32)
    dst = edge_index[1].astype(jnp.int32)
    batch = batch.astype(jnp.int32)
    ones_win = jnp.ones((DEG_WIN,), jnp.float32)
    zeros_pad = jnp.zeros((N_PAD,), jnp.float32)
    zeros_rows = jnp.zeros((N_PAD, D), jnp.float32)

    pad_idx = (N_NODES + (jnp.arange(NW * PAD_E, dtype=jnp.int32)
                          % (N_PAD - N_NODES))).reshape(NW, PAD_E)
    src = jnp.concatenate([src.reshape(NW, EDGES_PER_TILE), pad_idx],
                          axis=1).reshape(-1)
    dst = jnp.concatenate([dst.reshape(NW, EDGES_PER_TILE), pad_idx],
                          axis=1).reshape(-1)

    degp = _sc_deg(dst, ones_win, zeros_pad)

    g0, dis = pl.pallas_call(
        _tc_prep_body,
        out_shape=[
            jax.ShapeDtypeStruct((N_PAD, D), jnp.float32),
            jax.ShapeDtypeStruct((1, N_NODES), jnp.float32),
        ],
    )(x, W_gcn0, degp)

    acc0 = _sc_agg(g0, src, dst, zeros_rows)

    l1, g1 = pl.pallas_call(
        _tc_mid_body,
        out_shape=[
            jax.ShapeDtypeStruct((N_NODES, D), jnp.float32),
            jax.ShapeDtypeStruct((N_PAD, D), jnp.float32),
        ],
    )(acc0, g0, dis, b_gcn0.reshape(1, D), W_gcn1)

    acc1 = _sc_agg(g1, src, dst, zeros_rows)

    outs = pl.pallas_call(
        _tc_head_body,
        out_shape=[
            jax.ShapeDtypeStruct((1, B), jnp.float32),
            jax.ShapeDtypeStruct((1, 1), jnp.float32),
            jax.ShapeDtypeStruct((1, 1), jnp.float32),
            jax.ShapeDtypeStruct((1, 1), jnp.float32),
            jax.ShapeDtypeStruct((1, 1), jnp.float32),
            jax.ShapeDtypeStruct((1, 1), jnp.float32),
            jax.ShapeDtypeStruct((B, D), jnp.float32),
        ],
    )(acc1, g1, dis, b_gcn1.reshape(1, D), l1,
      batch.reshape(1, N_NODES), pair_feature, y.reshape(1, B),
      W_enc, b_enc.reshape(1, -1), W_lin1, b_lin1.reshape(1, -1),
      W_lin2, b_lin2.reshape(1, -1), W_log1, b_log1.reshape(1, -1),
      W_log2, b_log2.reshape(1, -1))

    logits, loss, acc, loss_1, loss_2, loss_3, hidden2 = outs
    return (logits.reshape(B), loss.reshape(()), acc.reshape(()),
            loss_1.reshape(()), loss_2.reshape(()), loss_3.reshape(()),
            hidden2)


# speedup vs baseline: 34.4833x; 1.0880x over previous
import functools

import jax
import jax.numpy as jnp
from jax import lax
from jax.experimental import pallas as pl
from jax.experimental.pallas import tpu as pltpu
from jax.experimental.pallas import tpu_sc as plsc

N_NODES = 10000
N_EDGES = 320000
D_FEAT = 128
D = 128
B = 64
ALPHA = 0.5
BETA = 0.5

NC = 2
NS = 16
NW = NC * NS
EDGES_PER_TILE = N_EDGES // NW
PAD_E = 248
EPT_P = EDGES_PER_TILE + PAD_E
AGG_WIN = 168
AGG_NWIN = EPT_P // AGG_WIN
DEG_WIN = 1464
DEG_NWIN = EPT_P // DEG_WIN
N_PAD = 10240
ROWS_PER_TILE = N_NODES // NS
PAD_PER_TILE = N_PAD // NS


def _sc_mesh():
    return plsc.VectorSubcoreMesh(core_axis_name="c", subcore_axis_name="s",
                                  num_cores=NC, num_subcores=NS)


@functools.partial(
    pl.kernel,
    out_type=jax.ShapeDtypeStruct((NC, N_PAD), jnp.float32),
    mesh=_sc_mesh(),
    scratch_types=[
        pltpu.VMEM((DEG_WIN,), jnp.int32),
        pltpu.VMEM((DEG_WIN,), jnp.float32),
        pltpu.VMEM_SHARED((N_PAD,), jnp.float32),
        pltpu.SemaphoreType.DMA,
    ],
)
def _sc_deg(dst_hbm, ones_hbm, zeros_hbm, out_hbm, dst_v, ones_v, acc_sh, sem):
    cid = lax.axis_index("c")
    sid = lax.axis_index("s")
    wid = sid * NC + cid
    zbase = sid * PAD_PER_TILE
    pltpu.sync_copy(zeros_hbm.at[pl.ds(zbase, PAD_PER_TILE)],
                    acc_sh.at[pl.ds(zbase, PAD_PER_TILE)])
    pltpu.sync_copy(ones_hbm, ones_v)
    plsc.subcore_barrier()
    e_base = wid * EPT_P

    def body(w, _):
        off = e_base + w * DEG_WIN
        pltpu.sync_copy(dst_hbm.at[pl.ds(off, DEG_WIN)], dst_v)
        pltpu.sync_copy(ones_v, acc_sh.at[dst_v], add=True)
        return 0

    lax.fori_loop(0, DEG_NWIN, body, 0)
    plsc.subcore_barrier()
    pltpu.sync_copy(acc_sh.at[pl.ds(zbase, PAD_PER_TILE)],
                    out_hbm.at[cid, pl.ds(zbase, PAD_PER_TILE)])


@functools.partial(
    pl.kernel,
    out_type=jax.ShapeDtypeStruct((NC, N_PAD, D), jnp.float32),
    mesh=_sc_mesh(),
    scratch_types=[
        pltpu.VMEM((AGG_WIN,), jnp.int32),
        pltpu.VMEM((AGG_WIN,), jnp.int32),
        pltpu.VMEM((AGG_WIN,), jnp.int32),
        pltpu.VMEM((AGG_WIN,), jnp.int32),
        pltpu.VMEM((AGG_WIN, D), jnp.float32),
        pltpu.VMEM((AGG_WIN, D), jnp.float32),
        pltpu.VMEM_SHARED((N_PAD, D), jnp.float32),
        pltpu.SemaphoreType.DMA,
        pltpu.SemaphoreType.DMA,
        pltpu.SemaphoreType.DMA,
        pltpu.SemaphoreType.DMA,
        pltpu.SemaphoreType.DMA,
        pltpu.SemaphoreType.DMA,
    ],
)
def _sc_agg(g_hbm, src_hbm, dst_hbm, zrows_hbm, out_hbm,
            src0_v, dst0_v, src1_v, dst1_v, rows0_v, rows1_v, acc_sh,
            g0sem, g1sem, s0sem, s1sem, d0sem, d1sem):
    cid = lax.axis_index("c")
    sid = lax.axis_index("s")
    wid = sid * NC + cid
    rbase = sid * PAD_PER_TILE
    pltpu.sync_copy(zrows_hbm.at[pl.ds(rbase, PAD_PER_TILE)],
                    acc_sh.at[pl.ds(rbase, PAD_PER_TILE)])
    plsc.subcore_barrier()
    e_base = wid * EPT_P

    def eoff(w):
        return e_base + w * AGG_WIN

    pltpu.async_copy(dst_hbm.at[pl.ds(eoff(0), AGG_WIN)], dst0_v, d0sem)
    pltpu.sync_copy(src_hbm.at[pl.ds(eoff(0), AGG_WIN)], src0_v)
    pltpu.async_copy(g_hbm.at[src0_v], rows0_v, g0sem)
    pltpu.async_copy(src_hbm.at[pl.ds(eoff(1), AGG_WIN)], src1_v, s1sem)
    pltpu.async_copy(dst_hbm.at[pl.ds(eoff(1), AGG_WIN)], dst1_v, d1sem)

    def body(p, _):
        wa = 2 * p + 1
        pltpu.make_async_copy(src_hbm.at[pl.ds(eoff(wa), AGG_WIN)],
                              src1_v, s1sem).wait()
        pltpu.async_copy(g_hbm.at[src1_v], rows1_v, g1sem)
        pltpu.make_async_copy(g_hbm.at[src0_v], rows0_v, g0sem).wait()
        pltpu.make_async_copy(dst_hbm.at[pl.ds(eoff(wa - 1), AGG_WIN)],
                              dst0_v, d0sem).wait()
        pltpu.async_copy(src_hbm.at[pl.ds(eoff(wa + 1), AGG_WIN)],
                         src0_v, s0sem)
        pltpu.sync_copy(rows0_v, acc_sh.at[dst0_v], add=True)
        pltpu.async_copy(dst_hbm.at[pl.ds(eoff(wa + 1), AGG_WIN)],
                         dst0_v, d0sem)
        pltpu.make_async_copy(src_hbm.at[pl.ds(eoff(wa + 1), AGG_WIN)],
                              src0_v, s0sem).wait()
        pltpu.async_copy(g_hbm.at[src0_v], rows0_v, g0sem)
        pltpu.make_async_copy(g_hbm.at[src1_v], rows1_v, g1sem).wait()
        pltpu.make_async_copy(dst_hbm.at[pl.ds(eoff(wa), AGG_WIN)],
                              dst1_v, d1sem).wait()
        wnext = jnp.where(wa + 2 < AGG_NWIN, wa + 2, 0)
        pltpu.async_copy(src_hbm.at[pl.ds(eoff(wnext), AGG_WIN)],
                         src1_v, s1sem)
        pltpu.sync_copy(rows1_v, acc_sh.at[dst1_v], add=True)
        pltpu.async_copy(dst_hbm.at[pl.ds(eoff(wnext), AGG_WIN)],
                         dst1_v, d1sem)
        return 0

    lax.fori_loop(0, (AGG_NWIN - 1) // 2, body, 0)
    pltpu.make_async_copy(g_hbm.at[src0_v], rows0_v, g0sem).wait()
    pltpu.make_async_copy(dst_hbm.at[pl.ds(eoff(AGG_NWIN - 1), AGG_WIN)],
                          dst0_v, d0sem).wait()
    pltpu.sync_copy(rows0_v, acc_sh.at[dst0_v], add=True)
    pltpu.make_async_copy(src_hbm.at[pl.ds(eoff(0), AGG_WIN)],
                          src1_v, s1sem).wait()
    pltpu.make_async_copy(dst_hbm.at[pl.ds(eoff(0), AGG_WIN)],
                          dst1_v, d1sem).wait()
    plsc.subcore_barrier()
    pltpu.sync_copy(acc_sh.at[pl.ds(rbase, PAD_PER_TILE)],
                    out_hbm.at[cid, pl.ds(rbase, PAD_PER_TILE)])


def _tc_prep_body(x_ref, w_ref, deg_ref, g_ref, dis_ref):
    deg = deg_ref[0, :N_NODES] + deg_ref[1, :N_NODES] + 1.0
    dis = lax.rsqrt(deg)
    dis_ref[0, :] = dis
    h = jnp.dot(x_ref[...], w_ref[...], preferred_element_type=jnp.float32)
    g_ref[0:N_NODES, :] = h * dis[:, None]


def _tc_mid_body(acc_ref, g0_ref, dis_ref, b0_ref, w1_ref,
                 l1_ref, g1_ref):
    dis = dis_ref[0, :]
    agg = (acc_ref[0, :N_NODES, :] + acc_ref[1, :N_NODES, :]
           + g0_ref[:N_NODES, :])
    l1 = jnp.tanh(agg * dis[:, None] + b0_ref[...])
    l1_ref[...] = l1
    h1 = jnp.dot(l1, w1_ref[...], preferred_element_type=jnp.float32)
    g1_ref[0:N_NODES, :] = h1 * dis[:, None]


def _bce_terms(p, t):
    p = jnp.clip(p, 1e-7, 1.0 - 1e-7)
    return -jnp.mean(t * jnp.log(p) + (1.0 - t) * jnp.log(1.0 - p))


def _tc_head_body(acc_ref, g1_ref, dis_ref, b1_ref, l1_ref,
                  batch_ref, pf_ref, y_ref,
                  wenc_ref, benc_ref, wlin1_ref, blin1_ref, wlin2_ref,
                  blin2_ref, wlog1_ref, blog1_ref, wlog2_ref, blog2_ref,
                  logits_ref, loss_ref, acc_out_ref, l1o_ref, l2o_ref,
                  l3o_ref, hidden2_ref):
    dis = dis_ref[0, :]
    agg = (acc_ref[0, :N_NODES, :] + acc_ref[1, :N_NODES, :]
           + g1_ref[:N_NODES, :])
    l2 = jnp.tanh(agg * dis[:, None] + b1_ref[...])

    batch = batch_ref[0, :]
    brange = lax.broadcasted_iota(jnp.int32, (B, N_NODES), 0)
    idx = jnp.sum((batch[None, :] < brange).astype(jnp.int32), axis=1)
    node_iota = lax.broadcasted_iota(jnp.int32, (B, N_NODES), 1)
    onehot = (node_iota == idx[:, None]).astype(jnp.float32)
    sel1 = jnp.dot(onehot, l1_ref[...], preferred_element_type=jnp.float32)
    sel2 = jnp.dot(onehot, l2, preferred_element_type=jnp.float32)
    layer2 = jnp.concatenate([sel1, sel2], axis=1)

    hidden1 = jnp.tanh(
        jnp.dot(pf_ref[...], wenc_ref[...], preferred_element_type=jnp.float32)
        + benc_ref[...])
    hidden2 = jnp.dot(layer2, wlin1_ref[...],
                      preferred_element_type=jnp.float32) + blin1_ref[...]
    hidden2_ref[...] = hidden2
    hidden = jax.nn.relu(jnp.concatenate([hidden1, hidden2], axis=1))
    h1 = jax.nn.relu(hidden1)
    h2 = jax.nn.relu(hidden2)
    y = y_ref[0, :]
    logits1 = jax.nn.sigmoid(
        jnp.dot(h1, wlog1_ref[...], preferred_element_type=jnp.float32)
        + blog1_ref[...])[:, 0]
    logits2 = jax.nn.sigmoid(
        jnp.dot(h2, wlog2_ref[...], preferred_element_type=jnp.float32)
        + blog2_ref[...])[:, 0]
    logits = jax.nn.sigmoid(
        jnp.dot(hidden, wlin2_ref[...], preferred_element_type=jnp.float32)
        + blin2_ref[...])[:, 0]
    loss_1 = _bce_terms(logits1, y)
    loss_2 = _bce_terms(logits2, y)
    loss_3 = _bce_terms(logits, y)
    logits_ref[...] = logits.reshape(1, B)
    l1o_ref[...] = loss_1.reshape(1, 1)
    l2o_ref[...] = loss_2.reshape(1, 1)
    l3o_ref[...] = loss_3.reshape(1, 1)
    loss_ref[...] = (ALPHA * loss_1 + BETA * loss_2 + loss_3).reshape(1, 1)
    pred = (logits > 0.7).astype(jnp.float32)
    acc_out_ref[...] = jnp.mean((pred == y).astype(jnp.float32)).reshape(1, 1)


def kernel(x, edge_index, batch, pair_feature, y, W_gcn0, b_gcn0, W_gcn1,
           b_gcn1, W_enc, b_enc, W_lin1, b_lin1, W_lin2, b_lin2, W_log1,
           b_log1, W_log2, b_log2):
    src = edge_index[0].astype(jnp.int32)
    dst = edge_index[1].astype(jnp.int32)
    batch = batch.astype(jnp.int32)
    ones_win = jnp.ones((DEG_WIN,), jnp.float32)
    zeros_pad = jnp.zeros((N_PAD,), jnp.float32)
    zeros_rows = jnp.zeros((N_PAD, D), jnp.float32)

    pad_idx = (N_NODES + (jnp.arange(NW * PAD_E, dtype=jnp.int32)
                          % (N_PAD - N_NODES))).reshape(NW, PAD_E)
    src = jnp.concatenate([src.reshape(NW, EDGES_PER_TILE), pad_idx],
                          axis=1).reshape(-1)
    dst = jnp.concatenate([dst.reshape(NW, EDGES_PER_TILE), pad_idx],
                          axis=1).reshape(-1)

    degp = _sc_deg(dst, ones_win, zeros_pad)

    g0, dis = pl.pallas_call(
        _tc_prep_body,
        out_shape=[
            jax.ShapeDtypeStruct((N_PAD, D), jnp.float32),
            jax.ShapeDtypeStruct((1, N_NODES), jnp.float32),
        ],
    )(x, W_gcn0, degp)

    acc0 = _sc_agg(g0, src, dst, zeros_rows)

    l1, g1 = pl.pallas_call(
        _tc_mid_body,
        out_shape=[
            jax.ShapeDtypeStruct((N_NODES, D), jnp.float32),
            jax.ShapeDtypeStruct((N_PAD, D), jnp.float32),
        ],
    )(acc0, g0, dis, b_gcn0.reshape(1, D), W_gcn1)

    acc1 = _sc_agg(g1, src, dst, zeros_rows)

    outs = pl.pallas_call(
        _tc_head_body,
        out_shape=[
            jax.ShapeDtypeStruct((1, B), jnp.float32),
            jax.ShapeDtypeStruct((1, 1), jnp.float32),
            jax.ShapeDtypeStruct((1, 1), jnp.float32),
            jax.ShapeDtypeStruct((1, 1), jnp.float32),
            jax.ShapeDtypeStruct((1, 1), jnp.float32),
            jax.ShapeDtypeStruct((1, 1), jnp.float32),
            jax.ShapeDtypeStruct((B, D), jnp.float32),
        ],
    )(acc1, g1, dis, b_gcn1.reshape(1, D), l1,
      batch.reshape(1, N_NODES), pair_feature, y.reshape(1, B),
      W_enc, b_enc.reshape(1, -1), W_lin1, b_lin1.reshape(1, -1),
      W_lin2, b_lin2.reshape(1, -1), W_log1, b_log1.reshape(1, -1),
      W_log2, b_log2.reshape(1, -1))

    logits, loss, acc, loss_1, loss_2, loss_3, hidden2 = outs
    return (logits.reshape(B), loss.reshape(()), acc.reshape(()),
            loss_1.reshape(()), loss_2.reshape(()), loss_3.reshape(()),
            hidden2)


# speedup vs baseline: 34.4987x; 1.0004x over previous
import functools

import jax
import jax.numpy as jnp
from jax import lax
from jax.experimental import pallas as pl
from jax.experimental.pallas import tpu as pltpu
from jax.experimental.pallas import tpu_sc as plsc

N_NODES = 10000
N_EDGES = 320000
D_FEAT = 128
D = 128
B = 64
ALPHA = 0.5
BETA = 0.5

NC = 2
NS = 16
NW = NC * NS
EDGES_PER_TILE = N_EDGES // NW
PAD_E = 248
EPT_P = EDGES_PER_TILE + PAD_E
AGG_WIN = 168
AGG_NWIN = EPT_P // AGG_WIN
DEG_WIN = 1464
DEG_NWIN = EPT_P // DEG_WIN
N_PAD = 10240
PAD_PER_TILE = N_PAD // NS


def _sc_mesh():
    return plsc.VectorSubcoreMesh(core_axis_name="c", subcore_axis_name="s",
                                  num_cores=NC, num_subcores=NS)


@functools.partial(
    pl.kernel,
    out_type=jax.ShapeDtypeStruct((NC, N_PAD), jnp.float32),
    mesh=_sc_mesh(),
    scratch_types=[
        pltpu.VMEM((DEG_WIN,), jnp.int32),
        pltpu.VMEM((DEG_WIN,), jnp.float32),
        pltpu.VMEM_SHARED((N_PAD,), jnp.float32),
        pltpu.SemaphoreType.DMA,
    ],
)
def _sc_deg(dst_hbm, ones_hbm, zeros_hbm, out_hbm, dst_v, ones_v, acc_sh, sem):
    cid = lax.axis_index("c")
    sid = lax.axis_index("s")
    wid = sid * NC + cid
    zbase = sid * PAD_PER_TILE
    pltpu.sync_copy(zeros_hbm.at[pl.ds(zbase, PAD_PER_TILE)],
                    acc_sh.at[pl.ds(zbase, PAD_PER_TILE)])
    pltpu.sync_copy(ones_hbm, ones_v)
    plsc.subcore_barrier()
    e_base = wid * EPT_P

    def body(w, _):
        off = e_base + w * DEG_WIN
        pltpu.sync_copy(dst_hbm.at[pl.ds(off, DEG_WIN)], dst_v)
        pltpu.sync_copy(ones_v, acc_sh.at[dst_v], add=True)
        return 0

    lax.fori_loop(0, DEG_NWIN, body, 0)
    plsc.subcore_barrier()
    pltpu.sync_copy(acc_sh.at[pl.ds(zbase, PAD_PER_TILE)],
                    out_hbm.at[cid, pl.ds(zbase, PAD_PER_TILE)])


@functools.partial(
    pl.kernel,
    out_type=jax.ShapeDtypeStruct((NC, N_PAD, D), jnp.float32),
    mesh=_sc_mesh(),
    scratch_types=[
        pltpu.VMEM((AGG_WIN,), jnp.int32),
        pltpu.VMEM((AGG_WIN,), jnp.int32),
        pltpu.VMEM((AGG_WIN,), jnp.int32),
        pltpu.VMEM((AGG_WIN,), jnp.int32),
        pltpu.VMEM((AGG_WIN, D), jnp.float32),
        pltpu.VMEM((AGG_WIN, D), jnp.float32),
        pltpu.VMEM_SHARED((N_PAD, D), jnp.float32),
        pltpu.SemaphoreType.DMA,
        pltpu.SemaphoreType.DMA,
        pltpu.SemaphoreType.DMA,
        pltpu.SemaphoreType.DMA,
        pltpu.SemaphoreType.DMA,
        pltpu.SemaphoreType.DMA,
    ],
)
def _sc_agg(g_hbm, src_hbm, dst_hbm, zrows_hbm, out_hbm,
            src0_v, dst0_v, src1_v, dst1_v, rows0_v, rows1_v, acc_sh,
            g0sem, g1sem, s0sem, s1sem, d0sem, d1sem):
    cid = lax.axis_index("c")
    sid = lax.axis_index("s")
    wid = sid * NC + cid
    rbase = sid * PAD_PER_TILE
    pltpu.sync_copy(zrows_hbm.at[pl.ds(rbase, PAD_PER_TILE)],
                    acc_sh.at[pl.ds(rbase, PAD_PER_TILE)])
    plsc.subcore_barrier()
    e_base = wid * EPT_P

    def eoff(w):
        return e_base + w * AGG_WIN

    pltpu.async_copy(dst_hbm.at[pl.ds(eoff(0), AGG_WIN)], dst0_v, d0sem)
    pltpu.sync_copy(src_hbm.at[pl.ds(eoff(0), AGG_WIN)], src0_v)
    pltpu.async_copy(g_hbm.at[src0_v], rows0_v, g0sem)
    pltpu.async_copy(src_hbm.at[pl.ds(eoff(1), AGG_WIN)], src1_v, s1sem)
    pltpu.async_copy(dst_hbm.at[pl.ds(eoff(1), AGG_WIN)], dst1_v, d1sem)

    def body(p, _):
        wa = 2 * p + 1
        pltpu.make_async_copy(src_hbm.at[pl.ds(eoff(wa), AGG_WIN)],
                              src1_v, s1sem).wait()
        pltpu.async_copy(g_hbm.at[src1_v], rows1_v, g1sem)
        pltpu.make_async_copy(g_hbm.at[src0_v], rows0_v, g0sem).wait()
        pltpu.make_async_copy(dst_hbm.at[pl.ds(eoff(wa - 1), AGG_WIN)],
                              dst0_v, d0sem).wait()
        pltpu.async_copy(src_hbm.at[pl.ds(eoff(wa + 1), AGG_WIN)],
                         src0_v, s0sem)
        pltpu.sync_copy(rows0_v, acc_sh.at[dst0_v], add=True)
        pltpu.async_copy(dst_hbm.at[pl.ds(eoff(wa + 1), AGG_WIN)],
                         dst0_v, d0sem)
        pltpu.make_async_copy(src_hbm.at[pl.ds(eoff(wa + 1), AGG_WIN)],
                              src0_v, s0sem).wait()
        pltpu.async_copy(g_hbm.at[src0_v], rows0_v, g0sem)
        pltpu.make_async_copy(g_hbm.at[src1_v], rows1_v, g1sem).wait()
        pltpu.make_async_copy(dst_hbm.at[pl.ds(eoff(wa), AGG_WIN)],
                              dst1_v, d1sem).wait()
        wnext = jnp.where(wa + 2 < AGG_NWIN, wa + 2, 0)
        pltpu.async_copy(src_hbm.at[pl.ds(eoff(wnext), AGG_WIN)],
                         src1_v, s1sem)
        pltpu.sync_copy(rows1_v, acc_sh.at[dst1_v], add=True)
        pltpu.async_copy(dst_hbm.at[pl.ds(eoff(wnext), AGG_WIN)],
                         dst1_v, d1sem)
        return 0

    lax.fori_loop(0, (AGG_NWIN - 1) // 2, body, 0)
    pltpu.make_async_copy(g_hbm.at[src0_v], rows0_v, g0sem).wait()
    pltpu.make_async_copy(dst_hbm.at[pl.ds(eoff(AGG_NWIN - 1), AGG_WIN)],
                          dst0_v, d0sem).wait()
    pltpu.sync_copy(rows0_v, acc_sh.at[dst0_v], add=True)
    pltpu.make_async_copy(src_hbm.at[pl.ds(eoff(0), AGG_WIN)],
                          src1_v, s1sem).wait()
    pltpu.make_async_copy(dst_hbm.at[pl.ds(eoff(0), AGG_WIN)],
                          dst1_v, d1sem).wait()
    plsc.subcore_barrier()
    pltpu.sync_copy(acc_sh.at[pl.ds(rbase, PAD_PER_TILE)],
                    out_hbm.at[cid, pl.ds(rbase, PAD_PER_TILE)])


def _tc_prep_body(x_ref, w_ref, deg_ref, g_ref, dis_ref):
    deg = deg_ref[0, :N_NODES] + deg_ref[1, :N_NODES] + 1.0
    dis = lax.rsqrt(deg)
    dis_ref[0, :] = dis
    h = jnp.dot(x_ref[...], w_ref[...], preferred_element_type=jnp.float32)
    g_ref[0:N_NODES, :] = h * dis[:, None]


def _tc_mid_body(acc_ref, g0_ref, dis_ref, b0_ref, w1_ref,
                 l1_ref, g1_ref):
    dis = dis_ref[0, :]
    agg = (acc_ref[0, :N_NODES, :] + acc_ref[1, :N_NODES, :]
           + g0_ref[:N_NODES, :])
    l1 = jnp.tanh(agg * dis[:, None] + b0_ref[...])
    l1_ref[...] = l1
    h1 = jnp.dot(l1, w1_ref[...], preferred_element_type=jnp.float32)
    g1_ref[0:N_NODES, :] = h1 * dis[:, None]


def _bce_terms(p, t):
    p = jnp.clip(p, 1e-7, 1.0 - 1e-7)
    return -jnp.mean(t * jnp.log(p) + (1.0 - t) * jnp.log(1.0 - p))


def _tc_head_body(acc_ref, g1_ref, dis_ref, b1_ref, l1_ref,
                  batch_ref, pf_ref, y_ref,
                  wenc_ref, benc_ref, wlin1_ref, blin1_ref, wlin2_ref,
                  blin2_ref, wlog1_ref, blog1_ref, wlog2_ref, blog2_ref,
                  logits_ref, loss_ref, acc_out_ref, l1o_ref, l2o_ref,
                  l3o_ref, hidden2_ref):
    dis = dis_ref[0, :]
    agg = (acc_ref[0, :N_NODES, :] + acc_ref[1, :N_NODES, :]
           + g1_ref[:N_NODES, :])
    l2 = jnp.tanh(agg * dis[:, None] + b1_ref[...])

    batch = batch_ref[0, :]
    brange = lax.broadcasted_iota(jnp.int32, (B, N_NODES), 0)
    idx = jnp.sum((batch[None, :] < brange).astype(jnp.int32), axis=1)
    node_iota = lax.broadcasted_iota(jnp.int32, (B, N_NODES), 1)
    onehot = (node_iota == idx[:, None]).astype(jnp.float32)
    sel1 = jnp.dot(onehot, l1_ref[...], preferred_element_type=jnp.float32)
    sel2 = jnp.dot(onehot, l2, preferred_element_type=jnp.float32)
    layer2 = jnp.concatenate([sel1, sel2], axis=1)

    hidden1 = jnp.tanh(
        jnp.dot(pf_ref[...], wenc_ref[...], preferred_element_type=jnp.float32)
        + benc_ref[...])
    hidden2 = jnp.dot(layer2, wlin1_ref[...],
                      preferred_element_type=jnp.float32) + blin1_ref[...]
    hidden2_ref[...] = hidden2
    hidden = jax.nn.relu(jnp.concatenate([hidden1, hidden2], axis=1))
    h1 = jax.nn.relu(hidden1)
    h2 = jax.nn.relu(hidden2)
    y = y_ref[0, :]
    logits1 = jax.nn.sigmoid(
        jnp.dot(h1, wlog1_ref[...], preferred_element_type=jnp.float32)
        + blog1_ref[...])[:, 0]
    logits2 = jax.nn.sigmoid(
        jnp.dot(h2, wlog2_ref[...], preferred_element_type=jnp.float32)
        + blog2_ref[...])[:, 0]
    logits = jax.nn.sigmoid(
        jnp.dot(hidden, wlin2_ref[...], preferred_element_type=jnp.float32)
        + blin2_ref[...])[:, 0]
    loss_1 = _bce_terms(logits1, y)
    loss_2 = _bce_terms(logits2, y)
    loss_3 = _bce_terms(logits, y)
    logits_ref[...] = logits.reshape(1, B)
    l1o_ref[...] = loss_1.reshape(1, 1)
    l2o_ref[...] = loss_2.reshape(1, 1)
    l3o_ref[...] = loss_3.reshape(1, 1)
    loss_ref[...] = (ALPHA * loss_1 + BETA * loss_2 + loss_3).reshape(1, 1)
    pred = (logits > 0.7).astype(jnp.float32)
    acc_out_ref[...] = jnp.mean((pred == y).astype(jnp.float32)).reshape(1, 1)


def kernel(x, edge_index, batch, pair_feature, y, W_gcn0, b_gcn0, W_gcn1,
           b_gcn1, W_enc, b_enc, W_lin1, b_lin1, W_lin2, b_lin2, W_log1,
           b_log1, W_log2, b_log2):
    src = edge_index[0].astype(jnp.int32)
    dst = edge_index[1].astype(jnp.int32)
    batch = batch.astype(jnp.int32)
    ones_win = jnp.ones((DEG_WIN,), jnp.float32)
    zeros_pad = jnp.zeros((N_PAD,), jnp.float32)
    zeros_rows = jnp.zeros((N_PAD, D), jnp.float32)

    pad_idx = (N_NODES + (jnp.arange(NW * PAD_E, dtype=jnp.int32)
                          % (N_PAD - N_NODES))).reshape(NW, PAD_E)
    src = jnp.concatenate([src.reshape(NW, EDGES_PER_TILE), pad_idx],
                          axis=1).reshape(-1)
    dst = jnp.concatenate([dst.reshape(NW, EDGES_PER_TILE), pad_idx],
                          axis=1).reshape(-1)

    degp = _sc_deg(dst, ones_win, zeros_pad)

    g0, dis = pl.pallas_call(
        _tc_prep_body,
        out_shape=[
            jax.ShapeDtypeStruct((N_PAD, D), jnp.float32),
            jax.ShapeDtypeStruct((1, N_NODES), jnp.float32),
        ],
    )(x, W_gcn0, degp)

    acc0 = _sc_agg(g0, src, dst, zeros_rows)

    l1, g1 = pl.pallas_call(
        _tc_mid_body,
        out_shape=[
            jax.ShapeDtypeStruct((N_NODES, D), jnp.float32),
            jax.ShapeDtypeStruct((N_PAD, D), jnp.float32),
        ],
    )(acc0, g0, dis, b_gcn0.reshape(1, D), W_gcn1)

    acc1 = _sc_agg(g1, src, dst, zeros_rows)

    outs = pl.pallas_call(
        _tc_head_body,
        out_shape=[
            jax.ShapeDtypeStruct((1, B), jnp.float32),
            jax.ShapeDtypeStruct((1, 1), jnp.float32),
            jax.ShapeDtypeStruct((1, 1), jnp.float32),
            jax.ShapeDtypeStruct((1, 1), jnp.float32),
            jax.ShapeDtypeStruct((1, 1), jnp.float32),
            jax.ShapeDtypeStruct((1, 1), jnp.float32),
            jax.ShapeDtypeStruct((B, D), jnp.float32),
        ],
    )(acc1, g1, dis, b_gcn1.reshape(1, D), l1,
      batch.reshape(1, N_NODES), pair_feature, y.reshape(1, B),
      W_enc, b_enc.reshape(1, -1), W_lin1, b_lin1.reshape(1, -1),
      W_lin2, b_lin2.reshape(1, -1), W_log1, b_log1.reshape(1, -1),
      W_log2, b_log2.reshape(1, -1))

    logits, loss, acc, loss_1, loss_2, loss_3, hidden2 = outs
    return (logits.reshape(B), loss.reshape(()), acc.reshape(()),
            loss_1.reshape(()), loss_2.reshape(()), loss_3.reshape(()),
            hidden2)
